# Initial kernel scaffold; baseline (speedup 1.0000x reference)
import jax
import jax.numpy as jnp
from jax.experimental import pallas as pl


def kernel(inputs, is_training, embeddings, ema_cluster_size_hidden, ema_dw_hidden, counter):
    raise NotImplementedError("write your pallas kernel here")


# speedup vs baseline: 1.2959x; 1.2959x over previous
import functools

import jax
import jax.numpy as jnp
from jax import lax
from jax.experimental import pallas as pl
from jax.experimental.pallas import tpu as pltpu
from jax.experimental.pallas import tpu_sc as plsc

_D = 256
_K = 8192
_N = 4608
_T = 128
_G = _N // _T
_COMMIT = 0.25
_DECAY = 0.99
_EPS = 1e-05


def _vq_body(x_ref, e_ref, e2_ref, dist_ref, enc_ref, idx_ref,
             counts_ref, dw_ref, lsum_ref):
    i = pl.program_id(0)
    x = x_ref[...]
    mm = jnp.dot(x, e_ref[...], preferred_element_type=jnp.float32)
    x2 = jnp.sum(x * x, axis=1, keepdims=True)
    dist = (x2 - 2.0 * mm) + e2_ref[...]
    dist_ref[...] = dist

    mind = jnp.min(dist, axis=1, keepdims=True)
    colids = lax.broadcasted_iota(jnp.int32, dist.shape, 1)
    idx = jnp.min(jnp.where(dist == mind, colids, jnp.int32(_K)), axis=1)
    idx_ref[...] = idx.reshape(1, 1, _T)

    enc = (colids == idx[:, None]).astype(jnp.float32)
    enc_ref[...] = enc

    c = jnp.sum(enc, axis=0, keepdims=True)
    dwp = lax.dot_general(x, enc, (((0,), (0,)), ((), ())),
                          preferred_element_type=jnp.float32)
    lpart = jnp.sum(mind)

    @pl.when(i == 0)
    def _init():
        counts_ref[...] = c
        dw_ref[...] = dwp
        lsum_ref[0, 0] = lpart

    @pl.when(i != 0)
    def _acc():
        counts_ref[...] += c
        dw_ref[...] += dwp
        lsum_ref[0, 0] += lpart


def _vq_core(flat_x, embeddings, e2):
    return pl.pallas_call(
        _vq_body,
        grid=(_G,),
        in_specs=[
            pl.BlockSpec((_T, _D), lambda i: (i, 0)),
            pl.BlockSpec((_D, _K), lambda i: (0, 0)),
            pl.BlockSpec((1, _K), lambda i: (0, 0)),
        ],
        out_specs=[
            pl.BlockSpec((_T, _K), lambda i: (i, 0)),
            pl.BlockSpec((_T, _K), lambda i: (i, 0)),
            pl.BlockSpec((1, 1, _T), lambda i: (i, 0, 0)),
            pl.BlockSpec((1, _K), lambda i: (0, 0)),
            pl.BlockSpec((_D, _K), lambda i: (0, 0)),
            pl.BlockSpec(memory_space=pltpu.SMEM, block_shape=(1, 1),
                         index_map=lambda i: (0, 0)),
        ],
        out_shape=[
            jax.ShapeDtypeStruct((_N, _K), jnp.float32),
            jax.ShapeDtypeStruct((_N, _K), jnp.float32),
            jax.ShapeDtypeStruct((_G, 1, _T), jnp.int32),
            jax.ShapeDtypeStruct((1, _K), jnp.float32),
            jax.ShapeDtypeStruct((_D, _K), jnp.float32),
            jax.ShapeDtypeStruct((1, 1), jnp.float32),
        ],
    )(flat_x, embeddings, e2)


def _make_sc_gather():
    info = plsc.get_sparse_core_info()
    nc, ns = info.num_cores, info.num_subcores
    nw = nc * ns
    bpw = _N // nw
    mesh = plsc.VectorSubcoreMesh(core_axis_name="c", subcore_axis_name="s")

    @functools.partial(
        pl.kernel, mesh=mesh,
        out_type=jax.ShapeDtypeStruct((_N, _D), jnp.float32),
        scratch_types=[
            pltpu.VMEM((bpw,), jnp.int32),
            pltpu.VMEM((bpw, _D), jnp.float32),
            pltpu.SemaphoreType.DMA,
        ],
    )
    def gather_k(table_hbm, idx_hbm, out_hbm, idx_v, rows_v, sem):
        wid = lax.axis_index("s") * nc + lax.axis_index("c")
        base = wid * bpw
        pltpu.sync_copy(idx_hbm.at[pl.ds(base, bpw)], idx_v)
        pltpu.async_copy(table_hbm.at[idx_v], rows_v, sem).wait()
        pltpu.sync_copy(rows_v, out_hbm.at[pl.ds(base, bpw)])

    return gather_k


def kernel(inputs, is_training, embeddings, ema_cluster_size_hidden,
           ema_dw_hidden, counter):
    flat_x = inputs.reshape(_N, _D)
    e2 = jnp.sum(embeddings ** 2, axis=0, keepdims=True)

    distances, encodings, idx_blocks, counts2d, dw, lsum = _vq_core(
        flat_x, embeddings, e2)

    idx_flat = idx_blocks.reshape(_N)
    encoding_indices = idx_flat.reshape(inputs.shape[:-1])

    table = embeddings.T
    gathered = _make_sc_gather()(table, idx_flat)
    quantized = inputs + (gathered.reshape(inputs.shape) - inputs)

    counts = counts2d[0]
    c1 = counter + 1
    bias = 1.0 - jnp.power(_DECAY, c1.astype(jnp.float32))
    upd_cs = (ema_cluster_size_hidden * _DECAY + counts * (1.0 - _DECAY)) / bias
    upd_dw = (ema_dw_hidden * _DECAY + dw * (1.0 - _DECAY)) / bias
    n = jnp.sum(upd_cs)
    stable_cs = (upd_cs + _EPS) / (n + _K * _EPS) * n
    trained = upd_dw / stable_cs[None, :]
    new_embeddings = jnp.where(is_training, trained, embeddings)

    loss = _COMMIT * (lsum[0, 0] / (_N * _D))
    avg_probs = counts / _N
    perplexity = jnp.exp(-jnp.sum(avg_probs * jnp.log(avg_probs + 1e-10)))

    return (quantized, loss, perplexity, encodings, encoding_indices,
            distances, new_embeddings)


# speedup vs baseline: 1.3348x; 1.0300x over previous
import functools

import jax
import jax.numpy as jnp
from jax import lax
from jax.experimental import pallas as pl
from jax.experimental.pallas import tpu as pltpu
from jax.experimental.pallas import tpu_sc as plsc

_D = 256
_K = 8192
_N = 4608
_T = 128
_G = _N // _T
_COMMIT = 0.25
_DECAY = 0.99
_EPS = 1e-05


def _vq_body(x_ref, e_ref, e2_ref, dist_ref, enc_ref, idx_ref,
             counts_ref, dw_ref, lsum_ref):
    i = pl.program_id(0)
    x = x_ref[...]
    mm = jnp.dot(x, e_ref[...], preferred_element_type=jnp.float32)
    x2 = jnp.sum(x * x, axis=1, keepdims=True)
    dist = (x2 - 2.0 * mm) + e2_ref[...]
    dist_ref[...] = dist

    mind = jnp.min(dist, axis=1, keepdims=True)
    colids = lax.broadcasted_iota(jnp.int32, dist.shape, 1)
    idx = jnp.min(jnp.where(dist == mind, colids, jnp.int32(_K)), axis=1)
    idx_ref[...] = idx.reshape(1, 1, _T)

    enc = (colids == idx[:, None]).astype(jnp.float32)
    enc_ref[...] = enc

    c = jnp.sum(enc, axis=0, keepdims=True)
    dwp = lax.dot_general(x.astype(jnp.bfloat16), enc.astype(jnp.bfloat16),
                          (((0,), (0,)), ((), ())),
                          preferred_element_type=jnp.float32)
    lpart = jnp.sum(mind)

    @pl.when(i == 0)
    def _init():
        counts_ref[...] = c
        dw_ref[...] = dwp
        lsum_ref[0, 0] = lpart

    @pl.when(i != 0)
    def _acc():
        counts_ref[...] += c
        dw_ref[...] += dwp
        lsum_ref[0, 0] += lpart


def _vq_core(flat_x, embeddings, e2):
    return pl.pallas_call(
        _vq_body,
        grid=(_G,),
        in_specs=[
            pl.BlockSpec((_T, _D), lambda i: (i, 0)),
            pl.BlockSpec((_D, _K), lambda i: (0, 0)),
            pl.BlockSpec((1, _K), lambda i: (0, 0)),
        ],
        out_specs=[
            pl.BlockSpec((_T, _K), lambda i: (i, 0)),
            pl.BlockSpec((_T, _K), lambda i: (i, 0)),
            pl.BlockSpec((1, 1, _T), lambda i: (i, 0, 0)),
            pl.BlockSpec((1, _K), lambda i: (0, 0)),
            pl.BlockSpec((_D, _K), lambda i: (0, 0)),
            pl.BlockSpec(memory_space=pltpu.SMEM, block_shape=(1, 1),
                         index_map=lambda i: (0, 0)),
        ],
        out_shape=[
            jax.ShapeDtypeStruct((_N, _K), jnp.float32),
            jax.ShapeDtypeStruct((_N, _K), jnp.float32),
            jax.ShapeDtypeStruct((_G, 1, _T), jnp.int32),
            jax.ShapeDtypeStruct((1, _K), jnp.float32),
            jax.ShapeDtypeStruct((_D, _K), jnp.float32),
            jax.ShapeDtypeStruct((1, 1), jnp.float32),
        ],
    )(flat_x, embeddings, e2)


def _make_sc_gather():
    info = plsc.get_sparse_core_info()
    nc, ns = info.num_cores, info.num_subcores
    nw = nc * ns
    bpw = _N // nw
    mesh = plsc.VectorSubcoreMesh(core_axis_name="c", subcore_axis_name="s")

    @functools.partial(
        pl.kernel, mesh=mesh,
        out_type=jax.ShapeDtypeStruct((_N, _D), jnp.float32),
        scratch_types=[
            pltpu.VMEM((bpw,), jnp.int32),
            pltpu.VMEM((bpw, _D), jnp.float32),
            pltpu.SemaphoreType.DMA,
        ],
    )
    def gather_k(table_hbm, idx_hbm, out_hbm, idx_v, rows_v, sem):
        wid = lax.axis_index("s") * nc + lax.axis_index("c")
        base = wid * bpw
        pltpu.sync_copy(idx_hbm.at[pl.ds(base, bpw)], idx_v)
        pltpu.async_copy(table_hbm.at[idx_v], rows_v, sem).wait()
        pltpu.sync_copy(rows_v, out_hbm.at[pl.ds(base, bpw)])

    return gather_k


def kernel(inputs, is_training, embeddings, ema_cluster_size_hidden,
           ema_dw_hidden, counter):
    flat_x = inputs.reshape(_N, _D)
    e2 = jnp.sum(embeddings ** 2, axis=0, keepdims=True)

    distances, encodings, idx_blocks, counts2d, dw, lsum = _vq_core(
        flat_x, embeddings, e2)

    idx_flat = idx_blocks.reshape(_N)
    encoding_indices = idx_flat.reshape(inputs.shape[:-1])

    table = embeddings.T
    gathered = _make_sc_gather()(table, idx_flat)
    quantized = gathered.reshape(inputs.shape)

    counts = counts2d[0]
    c1 = counter + 1
    bias = 1.0 - jnp.power(_DECAY, c1.astype(jnp.float32))
    upd_cs = (ema_cluster_size_hidden * _DECAY + counts * (1.0 - _DECAY)) / bias
    upd_dw = (ema_dw_hidden * _DECAY + dw * (1.0 - _DECAY)) / bias
    n = jnp.sum(upd_cs)
    stable_cs = (upd_cs + _EPS) / (n + _K * _EPS) * n
    trained = upd_dw / stable_cs[None, :]
    new_embeddings = jnp.where(is_training, trained, embeddings)

    loss = _COMMIT * (lsum[0, 0] / (_N * _D))
    avg_probs = counts / _N
    perplexity = jnp.exp(-jnp.sum(avg_probs * jnp.log(avg_probs + 1e-10)))

    return (quantized, loss, perplexity, encodings, encoding_indices,
            distances, new_embeddings)
